# baseline (speedup 1.0000x reference)
import functools

import jax
import jax.numpy as jnp
from jax.experimental import pallas as pl
from jax.experimental.pallas import tpu as pltpu

EMBED_DIM = 4096
N_EXPERTS = 64
TOP_K = 8
BT = 512


def _router_body(nsteps, total_tokens, x_ref, w_ref, probs_ref, idx_ref,
                 loss_ref, accp_ref, accm_ref):
    i = pl.program_id(0)
    logits = jax.lax.dot_general(
        w_ref[...], x_ref[...], (((1,), (1,)), ((), ())),
        preferred_element_type=jnp.float32)

    iota_e = jax.lax.broadcasted_iota(jnp.int32, (N_EXPERTS, BT), 0)
    l = logits
    sel = jnp.zeros((N_EXPERTS, BT), jnp.bool_)
    ids = []
    m1 = None
    for k in range(TOP_K):
        m = jnp.max(l, axis=0, keepdims=True)
        if k == 0:
            m1 = m
        cand = jnp.where(l == m, iota_e, N_EXPERTS)
        idx = jnp.min(cand, axis=0, keepdims=True)
        pick = iota_e == idx
        sel = jnp.logical_or(sel, pick)
        ids.append(idx)
        l = jnp.where(pick, -jnp.inf, l)
    idx_t = jnp.concatenate(ids, axis=0)
    idx_ref[...] = idx_t.astype(jnp.float32).T.astype(jnp.int32)

    e = jnp.exp(logits - m1)
    e_sel = jnp.where(sel, e, 0.0)
    probs_ref[...] = (e_sel / jnp.sum(e_sel, axis=0, keepdims=True)).T

    pfull = e / jnp.sum(e, axis=0, keepdims=True)
    self_f = sel.astype(jnp.float32)

    @pl.when(i == 0)
    def _init():
        accp_ref[...] = pfull
        accm_ref[...] = self_f

    @pl.when(i > 0)
    def _acc():
        accp_ref[...] += pfull
        accm_ref[...] += self_f

    @pl.when(i == nsteps - 1)
    def _fin():
        ps = jnp.sum(accp_ref[...], axis=1)
        ms = jnp.sum(accm_ref[...], axis=1)
        scale = jnp.float32(N_EXPERTS) / jnp.float32(total_tokens * total_tokens)
        loss_ref[0, 0] = scale * jnp.sum(ps * ms)


def kernel(x, W_gate, W_noise):
    del W_noise
    B, S, D = x.shape
    T = B * S
    xf = x.reshape(T, D)
    nsteps = T // BT

    probs, idx, loss = pl.pallas_call(
        functools.partial(_router_body, nsteps, T),
        grid=(nsteps,),
        in_specs=[
            pl.BlockSpec((BT, D), lambda i: (i, 0)),
            pl.BlockSpec((N_EXPERTS, D), lambda i: (0, 0)),
        ],
        out_specs=[
            pl.BlockSpec((BT, N_EXPERTS), lambda i: (i, 0)),
            pl.BlockSpec((BT, TOP_K), lambda i: (i, 0)),
            pl.BlockSpec(memory_space=pltpu.SMEM),
        ],
        out_shape=[
            jax.ShapeDtypeStruct((T, N_EXPERTS), jnp.float32),
            jax.ShapeDtypeStruct((T, TOP_K), jnp.int32),
            jax.ShapeDtypeStruct((1, 1), jnp.float32),
        ],
        scratch_shapes=[
            pltpu.VMEM((N_EXPERTS, BT), jnp.float32),
            pltpu.VMEM((N_EXPERTS, BT), jnp.float32),
        ],
        compiler_params=pltpu.CompilerParams(
            dimension_semantics=("arbitrary",)),
    )(xf, W_gate)

    return (probs.reshape(B, S, N_EXPERTS),
            idx.reshape(B, S, TOP_K), loss.reshape(()))


# speedup vs baseline: 1.1458x; 1.1458x over previous
import functools

import jax
import jax.numpy as jnp
from jax.experimental import pallas as pl
from jax.experimental.pallas import tpu as pltpu

EMBED_DIM = 4096
N_EXPERTS = 64
TOP_K = 8
BT = 1024


def _router_body(nsteps, total_tokens, x_ref, w_ref, probs_ref, idx_ref,
                 loss_ref, accp_ref, accm_ref):
    i = pl.program_id(0)
    logits = jax.lax.dot_general(
        w_ref[...], x_ref[...], (((1,), (1,)), ((), ())),
        preferred_element_type=jnp.float32)

    iota_e = jax.lax.broadcasted_iota(jnp.int32, (N_EXPERTS, BT), 0)
    l = logits
    sel = jnp.zeros((N_EXPERTS, BT), jnp.bool_)
    ids = []
    m1 = None
    for k in range(TOP_K):
        m = jnp.max(l, axis=0, keepdims=True)
        if k == 0:
            m1 = m
        cand = jnp.where(l == m, iota_e, N_EXPERTS)
        idx = jnp.min(cand, axis=0, keepdims=True)
        pick = iota_e == idx
        sel = jnp.logical_or(sel, pick)
        ids.append(idx)
        l = jnp.where(pick, -jnp.inf, l)
    idx_ref[...] = jnp.concatenate(ids, axis=0)

    e = jnp.exp(logits - m1)
    e_sel = jnp.where(sel, e, 0.0)
    probs_ref[...] = e_sel / jnp.sum(e_sel, axis=0, keepdims=True)

    pfull = e / jnp.sum(e, axis=0, keepdims=True)
    self_f = sel.astype(jnp.float32)

    @pl.when(i == 0)
    def _init():
        accp_ref[...] = pfull
        accm_ref[...] = self_f

    @pl.when(i > 0)
    def _acc():
        accp_ref[...] += pfull
        accm_ref[...] += self_f

    @pl.when(i == nsteps - 1)
    def _fin():
        ps = jnp.sum(accp_ref[...], axis=1)
        ms = jnp.sum(accm_ref[...], axis=1)
        scale = jnp.float32(N_EXPERTS) / jnp.float32(total_tokens * total_tokens)
        loss_ref[0, 0] = scale * jnp.sum(ps * ms)


def kernel(x, W_gate, W_noise):
    del W_noise
    B, S, D = x.shape
    T = B * S
    xf = x.reshape(T, D)
    nsteps = T // BT

    probs_t, idx_t, loss = pl.pallas_call(
        functools.partial(_router_body, nsteps, T),
        grid=(nsteps,),
        in_specs=[
            pl.BlockSpec((BT, D), lambda i: (i, 0)),
            pl.BlockSpec((N_EXPERTS, D), lambda i: (0, 0)),
        ],
        out_specs=[
            pl.BlockSpec((N_EXPERTS, BT), lambda i: (0, i)),
            pl.BlockSpec((TOP_K, BT), lambda i: (0, i)),
            pl.BlockSpec(memory_space=pltpu.SMEM),
        ],
        out_shape=[
            jax.ShapeDtypeStruct((N_EXPERTS, T), jnp.float32),
            jax.ShapeDtypeStruct((TOP_K, T), jnp.int32),
            jax.ShapeDtypeStruct((1, 1), jnp.float32),
        ],
        scratch_shapes=[
            pltpu.VMEM((N_EXPERTS, BT), jnp.float32),
            pltpu.VMEM((N_EXPERTS, BT), jnp.float32),
        ],
        compiler_params=pltpu.CompilerParams(
            dimension_semantics=("arbitrary",)),
    )(xf, W_gate)

    return (probs_t.T.reshape(B, S, N_EXPERTS),
            idx_t.T.reshape(B, S, TOP_K), loss.reshape(()))
